# baseline (device time: 57031 ns/iter reference)
import jax
import jax.numpy as jnp
from jax import lax
from jax.experimental import pallas as pl
from jax.experimental.pallas import tpu as pltpu

N_DEV = 8
SQ = 512
D_MODEL = 1024
SKV = 2048
H_LOC = 8
DH = 128
SCALE = 0.08838834764831843
CHUNK = SQ // N_DEV

PARTS = (
    (0, 384, ("z", "y", "x")),
    (384, 384, ("y", "x", "z")),
    (768, 256, ("x", "z", "y")),
)


def kernel(x, Wq, Wo, K_ext, V_ext):
    def body(x_ref, wq_ref, wo_ref, k_hbm, v_hbm, out_ref,
             k_scr, v_scr, attn_scr, comm0, comm1, comm2,
             rs_send, rs_recv, ag_send, ag_recv, k_sems, v_sems):
        my_i = lax.axis_index("i")

        b0 = jnp.bitwise_and(my_i, 1)
        b1 = jnp.bitwise_and(my_i // 2, 1)
        b2 = jnp.bitwise_and(my_i // 4, 1)
        coord = {"x": jnp.bitwise_xor(b0, b1), "y": b1, "z": b2}
        mask = {"x": 1, "y": 3, "z": 4}

        barrier = pltpu.get_barrier_semaphore()
        for d in ("x", "y", "z"):
            pl.semaphore_signal(
                barrier, inc=1,
                device_id=(jnp.bitwise_xor(my_i, mask[d]),),
                device_id_type=pl.DeviceIdType.MESH)

        h0 = my_i * H_LOC

        def kv_copy(pair, slot):
            k_cp = pltpu.make_async_copy(
                k_hbm.at[0, :, pl.ds(h0 + 2 * pair, 2), :], k_scr.at[slot],
                k_sems.at[slot])
            v_cp = pltpu.make_async_copy(
                v_hbm.at[0, :, pl.ds(h0 + 2 * pair, 2), :], v_scr.at[slot],
                v_sems.at[slot])
            return k_cp, v_cp

        k_cp, v_cp = kv_copy(0, 0)
        k_cp.start()
        v_cp.start()

        q = jnp.dot(x_ref[0] * SCALE, wq_ref[...],
                    preferred_element_type=jnp.float32)

        for pair in range(H_LOC // 2):
            slot = pair % 2
            k_cp, v_cp = kv_copy(pair, slot)
            k_cp.wait()
            v_cp.wait()
            if pair + 1 < H_LOC // 2:
                nk, nv = kv_copy(pair + 1, (pair + 1) % 2)
                nk.start()
                nv.start()
            for hh in range(2):
                h = 2 * pair + hh
                qh = q[:, h * DH:(h + 1) * DH]
                s = lax.dot_general(
                    qh, k_scr[slot, :, hh, :], (((1,), (1,)), ((), ())),
                    preferred_element_type=jnp.float32)
                p = jnp.exp(s)
                l = jnp.sum(p, axis=1, keepdims=True)
                o = lax.dot_general(
                    p, v_scr[slot, :, hh, :], (((1,), (0,)), ((), ())),
                    preferred_element_type=jnp.float32)
                attn_scr[:, h * DH:(h + 1) * DH] = o / l

        pl.semaphore_wait(barrier, 3)

        comms = [comm0, comm1, comm2]
        halves = (SQ // 2, SQ // 4, SQ // 8)
        rs_row = (0, 256, 384)
        ag_row = (448, 512, 640)

        half0 = halves[0]
        half0_c = half0 // CHUNK
        offs = []
        r0 = []
        for p, (c0, w, dims) in enumerate(PARTS):
            d = dims[0]
            partner = jnp.bitwise_xor(my_i, mask[d])
            lower = coord[d] == 0
            send_c = jnp.where(lower, half0_c, 0)
            keep_c = jnp.where(lower, 0, half0_c)
            rows = pl.ds(send_c * CHUNK, half0)
            out_ref[0, rows, pl.ds(c0, w)] = jnp.dot(
                attn_scr[rows, :], wo_ref[:, c0:c0 + w],
                preferred_element_type=jnp.float32)
            rdma = pltpu.make_async_remote_copy(
                src_ref=out_ref.at[0, rows, pl.ds(c0, w)],
                dst_ref=comms[p].at[pl.ds(rs_row[0], half0), :],
                send_sem=rs_send.at[3 * p],
                recv_sem=rs_recv.at[3 * p],
                device_id=(partner,),
                device_id_type=pl.DeviceIdType.MESH,
            )
            rdma.start()
            r0.append((rdma, keep_c))
            offs.append(keep_c)
        for p, (c0, w, dims) in enumerate(PARTS):
            rows = pl.ds(offs[p] * CHUNK, half0)
            out_ref[0, rows, pl.ds(c0, w)] = jnp.dot(
                attn_scr[rows, :], wo_ref[:, c0:c0 + w],
                preferred_element_type=jnp.float32)
        for p, (c0, w, dims) in enumerate(PARTS):
            rdma, keep_c = r0[p]
            rdma.wait()
            out_ref[0, pl.ds(keep_c * CHUNK, half0), pl.ds(c0, w)] = (
                out_ref[0, pl.ds(keep_c * CHUNK, half0), pl.ds(c0, w)]
                + comms[p][pl.ds(rs_row[0], half0), :])

        for r in range(1, 3):
            half = halves[r]
            half_c = half // CHUNK
            started = []
            for p, (c0, w, dims) in enumerate(PARTS):
                d = dims[r]
                partner = jnp.bitwise_xor(my_i, mask[d])
                lower = coord[d] == 0
                send_c = offs[p] + jnp.where(lower, half_c, 0)
                keep_c = offs[p] + jnp.where(lower, 0, half_c)
                rdma = pltpu.make_async_remote_copy(
                    src_ref=out_ref.at[0, pl.ds(send_c * CHUNK, half),
                                       pl.ds(c0, w)],
                    dst_ref=comms[p].at[pl.ds(rs_row[r], half), :],
                    send_sem=rs_send.at[3 * p + r],
                    recv_sem=rs_recv.at[3 * p + r],
                    device_id=(partner,),
                    device_id_type=pl.DeviceIdType.MESH,
                )
                rdma.start()
                started.append((rdma, keep_c))
                offs[p] = keep_c
            for p, (c0, w, dims) in enumerate(PARTS):
                rdma, keep_c = started[p]
                rdma.wait()
                out_ref[0, pl.ds(keep_c * CHUNK, half), pl.ds(c0, w)] = (
                    out_ref[0, pl.ds(keep_c * CHUNK, half), pl.ds(c0, w)]
                    + comms[p][pl.ds(rs_row[r], half), :])

        for j in range(3):
            blk = halves[2 - j]
            blk_c = blk // CHUNK
            started = []
            for p, (c0, w, dims) in enumerate(PARTS):
                d = dims[2 - j]
                partner = jnp.bitwise_xor(my_i, mask[d])
                lower = coord[d] == 0
                partner_c = jnp.where(lower, offs[p] + blk_c,
                                      offs[p] - blk_c)
                rdma = pltpu.make_async_remote_copy(
                    src_ref=out_ref.at[0, pl.ds(offs[p] * CHUNK, blk),
                                       pl.ds(c0, w)],
                    dst_ref=comms[p].at[pl.ds(ag_row[j], blk), :],
                    send_sem=ag_send.at[3 * p + j],
                    recv_sem=ag_recv.at[3 * p + j],
                    device_id=(partner,),
                    device_id_type=pl.DeviceIdType.MESH,
                )
                rdma.start()
                started.append((rdma, partner_c))
                offs[p] = jnp.where(lower, offs[p], offs[p] - blk_c)
            for p, (c0, w, dims) in enumerate(PARTS):
                rdma, partner_c = started[p]
                rdma.wait()
                out_ref[0, pl.ds(partner_c * CHUNK, blk), pl.ds(c0, w)] = \
                    comms[p][pl.ds(ag_row[j], blk), :]

    return pl.pallas_call(
        body,
        out_shape=jax.ShapeDtypeStruct((1, SQ, D_MODEL), jnp.float32),
        in_specs=[
            pl.BlockSpec(memory_space=pltpu.VMEM),
            pl.BlockSpec(memory_space=pltpu.VMEM),
            pl.BlockSpec(memory_space=pltpu.VMEM),
            pl.BlockSpec(memory_space=pl.ANY),
            pl.BlockSpec(memory_space=pl.ANY),
        ],
        out_specs=pl.BlockSpec(memory_space=pltpu.VMEM),
        scratch_shapes=[
            pltpu.VMEM((2, SKV, 2, DH), jnp.float32),
            pltpu.VMEM((2, SKV, 2, DH), jnp.float32),
            pltpu.VMEM((SQ, H_LOC * DH), jnp.float32),
            pltpu.VMEM((896, 384), jnp.float32),
            pltpu.VMEM((896, 384), jnp.float32),
            pltpu.VMEM((896, 256), jnp.float32),
            pltpu.SemaphoreType.DMA((9,)),
            pltpu.SemaphoreType.DMA((9,)),
            pltpu.SemaphoreType.DMA((9,)),
            pltpu.SemaphoreType.DMA((9,)),
            pltpu.SemaphoreType.DMA((2,)),
            pltpu.SemaphoreType.DMA((2,)),
        ],
        compiler_params=pltpu.CompilerParams(
            collective_id=0, vmem_limit_bytes=100 * 1024 * 1024),
    )(x, Wq, Wo, K_ext, V_ext)


# device time: 55660 ns/iter; 1.0246x vs baseline; 1.0246x over previous
import jax
import jax.numpy as jnp
from jax import lax
from jax.experimental import pallas as pl
from jax.experimental.pallas import tpu as pltpu

N_DEV = 8
SQ = 512
D_MODEL = 1024
SKV = 2048
H_LOC = 8
DH = 128
SCALE = 0.08838834764831843
CHUNK = SQ // N_DEV

PARTS = (
    (0, 384, ("z", "y", "x")),
    (384, 384, ("y", "x", "z")),
    (768, 256, ("x", "z", "y")),
)


def kernel(x, Wq, Wo, K_ext, V_ext):
    def body(x_ref, wq_ref, wo_ref, k_hbm, v_hbm, out_ref,
             k_scr, v_scr, attn_scr, comm0, comm1, comm2,
             rs_send, rs_recv, ag_send, ag_recv, k_sems, v_sems):
        my_i = lax.axis_index("i")

        b0 = jnp.bitwise_and(my_i, 1)
        b1 = jnp.bitwise_and(my_i // 2, 1)
        b2 = jnp.bitwise_and(my_i // 4, 1)
        coord = {"x": jnp.bitwise_xor(b0, b1), "y": b1, "z": b2}
        mask = {"x": 1, "y": 3, "z": 4}

        barrier = pltpu.get_barrier_semaphore()
        for d in ("x", "y", "z"):
            pl.semaphore_signal(
                barrier, inc=1,
                device_id=(jnp.bitwise_xor(my_i, mask[d]),),
                device_id_type=pl.DeviceIdType.MESH)

        h0 = my_i * H_LOC

        def kv_copy(h, slot):
            k_cp = pltpu.make_async_copy(
                k_hbm.at[0, :, pl.ds(h0 + h, 1), :], k_scr.at[slot],
                k_sems.at[slot])
            v_cp = pltpu.make_async_copy(
                v_hbm.at[0, :, pl.ds(h0 + h, 1), :], v_scr.at[slot],
                v_sems.at[slot])
            return k_cp, v_cp

        k_cp, v_cp = kv_copy(0, 0)
        k_cp.start()
        v_cp.start()

        q = jnp.dot(x_ref[0] * SCALE, wq_ref[...],
                    preferred_element_type=jnp.float32)

        for h in range(H_LOC):
            slot = h % 2
            k_cp, v_cp = kv_copy(h, slot)
            k_cp.wait()
            v_cp.wait()
            if h + 1 < H_LOC:
                nk, nv = kv_copy(h + 1, (h + 1) % 2)
                nk.start()
                nv.start()
            qh = q[:, h * DH:(h + 1) * DH]
            s = lax.dot_general(
                qh, k_scr[slot, :, 0, :], (((1,), (1,)), ((), ())),
                preferred_element_type=jnp.float32)
            p = jnp.exp(s)
            l = jnp.sum(p, axis=1, keepdims=True)
            o = lax.dot_general(
                p, v_scr[slot, :, 0, :], (((1,), (0,)), ((), ())),
                preferred_element_type=jnp.float32)
            attn_scr[:, h * DH:(h + 1) * DH] = o * (1.0 / l)

        pl.semaphore_wait(barrier, 3)

        comms = [comm0, comm1, comm2]
        halves = (SQ // 2, SQ // 4, SQ // 8)
        rs_row = (0, 256, 384)
        ag_row = (448, 512, 640)

        half0 = halves[0]
        half0_c = half0 // CHUNK
        offs = []
        r0 = []
        for p, (c0, w, dims) in enumerate(PARTS):
            d = dims[0]
            partner = jnp.bitwise_xor(my_i, mask[d])
            lower = coord[d] == 0
            send_c = jnp.where(lower, half0_c, 0)
            keep_c = jnp.where(lower, 0, half0_c)
            rows = pl.ds(send_c * CHUNK, half0)
            out_ref[0, rows, pl.ds(c0, w)] = jnp.dot(
                attn_scr[rows, :], wo_ref[:, c0:c0 + w],
                preferred_element_type=jnp.float32)
            rdma = pltpu.make_async_remote_copy(
                src_ref=out_ref.at[0, rows, pl.ds(c0, w)],
                dst_ref=comms[p].at[pl.ds(rs_row[0], half0), :],
                send_sem=rs_send.at[3 * p],
                recv_sem=rs_recv.at[3 * p],
                device_id=(partner,),
                device_id_type=pl.DeviceIdType.MESH,
            )
            rdma.start()
            r0.append((rdma, keep_c))
            offs.append(keep_c)
        for p, (c0, w, dims) in enumerate(PARTS):
            rows = pl.ds(offs[p] * CHUNK, half0)
            out_ref[0, rows, pl.ds(c0, w)] = jnp.dot(
                attn_scr[rows, :], wo_ref[:, c0:c0 + w],
                preferred_element_type=jnp.float32)
        for p, (c0, w, dims) in enumerate(PARTS):
            rdma, keep_c = r0[p]
            rdma.wait()
            out_ref[0, pl.ds(keep_c * CHUNK, half0), pl.ds(c0, w)] = (
                out_ref[0, pl.ds(keep_c * CHUNK, half0), pl.ds(c0, w)]
                + comms[p][pl.ds(rs_row[0], half0), :])

        for r in range(1, 3):
            half = halves[r]
            half_c = half // CHUNK
            started = []
            for p, (c0, w, dims) in enumerate(PARTS):
                d = dims[r]
                partner = jnp.bitwise_xor(my_i, mask[d])
                lower = coord[d] == 0
                send_c = offs[p] + jnp.where(lower, half_c, 0)
                keep_c = offs[p] + jnp.where(lower, 0, half_c)
                rdma = pltpu.make_async_remote_copy(
                    src_ref=out_ref.at[0, pl.ds(send_c * CHUNK, half),
                                       pl.ds(c0, w)],
                    dst_ref=comms[p].at[pl.ds(rs_row[r], half), :],
                    send_sem=rs_send.at[3 * p + r],
                    recv_sem=rs_recv.at[3 * p + r],
                    device_id=(partner,),
                    device_id_type=pl.DeviceIdType.MESH,
                )
                rdma.start()
                started.append((rdma, keep_c))
                offs[p] = keep_c
            for p, (c0, w, dims) in enumerate(PARTS):
                rdma, keep_c = started[p]
                rdma.wait()
                out_ref[0, pl.ds(keep_c * CHUNK, half), pl.ds(c0, w)] = (
                    out_ref[0, pl.ds(keep_c * CHUNK, half), pl.ds(c0, w)]
                    + comms[p][pl.ds(rs_row[r], half), :])

        for j in range(3):
            blk = halves[2 - j]
            blk_c = blk // CHUNK
            started = []
            for p, (c0, w, dims) in enumerate(PARTS):
                d = dims[2 - j]
                partner = jnp.bitwise_xor(my_i, mask[d])
                lower = coord[d] == 0
                partner_c = jnp.where(lower, offs[p] + blk_c,
                                      offs[p] - blk_c)
                rdma = pltpu.make_async_remote_copy(
                    src_ref=out_ref.at[0, pl.ds(offs[p] * CHUNK, blk),
                                       pl.ds(c0, w)],
                    dst_ref=comms[p].at[pl.ds(ag_row[j], blk), :],
                    send_sem=ag_send.at[3 * p + j],
                    recv_sem=ag_recv.at[3 * p + j],
                    device_id=(partner,),
                    device_id_type=pl.DeviceIdType.MESH,
                )
                rdma.start()
                started.append((rdma, partner_c))
                offs[p] = jnp.where(lower, offs[p], offs[p] - blk_c)
            for p, (c0, w, dims) in enumerate(PARTS):
                rdma, partner_c = started[p]
                rdma.wait()
                out_ref[0, pl.ds(partner_c * CHUNK, blk), pl.ds(c0, w)] = \
                    comms[p][pl.ds(ag_row[j], blk), :]

    return pl.pallas_call(
        body,
        out_shape=jax.ShapeDtypeStruct((1, SQ, D_MODEL), jnp.float32),
        in_specs=[
            pl.BlockSpec(memory_space=pltpu.VMEM),
            pl.BlockSpec(memory_space=pltpu.VMEM),
            pl.BlockSpec(memory_space=pltpu.VMEM),
            pl.BlockSpec(memory_space=pl.ANY),
            pl.BlockSpec(memory_space=pl.ANY),
        ],
        out_specs=pl.BlockSpec(memory_space=pltpu.VMEM),
        scratch_shapes=[
            pltpu.VMEM((2, SKV, 1, DH), jnp.float32),
            pltpu.VMEM((2, SKV, 1, DH), jnp.float32),
            pltpu.VMEM((SQ, H_LOC * DH), jnp.float32),
            pltpu.VMEM((896, 384), jnp.float32),
            pltpu.VMEM((896, 384), jnp.float32),
            pltpu.VMEM((896, 256), jnp.float32),
            pltpu.SemaphoreType.DMA((9,)),
            pltpu.SemaphoreType.DMA((9,)),
            pltpu.SemaphoreType.DMA((9,)),
            pltpu.SemaphoreType.DMA((9,)),
            pltpu.SemaphoreType.DMA((2,)),
            pltpu.SemaphoreType.DMA((2,)),
        ],
        compiler_params=pltpu.CompilerParams(
            collective_id=0, vmem_limit_bytes=100 * 1024 * 1024),
    )(x, Wq, Wo, K_ext, V_ext)
